# baseline (speedup 1.0000x reference)
import functools

import jax
import jax.numpy as jnp
from jax.experimental import pallas as pl

_TOP_K = 8
_NEG_INF = float("-inf")


_N_SUB = 4


def _gate_body(x_ref, w_ref, idx_ref, wgt_ref):
    w = w_ref[:]
    sb = x_ref.shape[0] // _N_SUB
    for s in range(_N_SUB):
        rows = pl.ds(s * sb, sb)
        logits = jax.lax.dot_general(
            x_ref[rows, :], w, (((1,), (1,)), ((), ())),
            preferred_element_type=jnp.float32,
        )
        topi, wgt = _topk_softmax(logits)
        idx_ref[rows, :] = topi
        wgt_ref[rows, :] = wgt


def _topk_softmax(logits):
    bt, e = logits.shape
    lanef = jax.lax.broadcasted_iota(jnp.int32, (bt, e), 1).astype(jnp.float32)

    vals = []
    idxs = []
    cur = logits
    for _ in range(_TOP_K):
        m = jnp.max(cur, axis=-1, keepdims=True)
        a = jnp.min(
            jnp.where(cur == m, lanef, jnp.float32(e)), axis=-1, keepdims=True
        )
        vals.append(m)
        idxs.append(a)
        cur = jnp.where(lanef == a, _NEG_INF, cur)

    topv = jnp.concatenate(vals, axis=-1)
    topi = jnp.concatenate(idxs, axis=-1).astype(jnp.int32)

    ex = jnp.exp(topv - topv[:, 0:1])
    wgt = ex / jnp.sum(ex, axis=-1, keepdims=True)
    return topi, wgt


@functools.partial(jax.jit, static_argnames=())
def _gate(flat, weight):
    t, h = flat.shape
    e = weight.shape[0]
    bt = 1024
    grid = (t // bt,)
    topi, topw = pl.pallas_call(
        _gate_body,
        grid=grid,
        in_specs=[
            pl.BlockSpec((bt, h), lambda i: (i, 0)),
            pl.BlockSpec((e, h), lambda i: (0, 0)),
        ],
        out_specs=[
            pl.BlockSpec((bt, _TOP_K), lambda i: (i, 0)),
            pl.BlockSpec((bt, _TOP_K), lambda i: (i, 0)),
        ],
        out_shape=[
            jax.ShapeDtypeStruct((t, _TOP_K), jnp.int32),
            jax.ShapeDtypeStruct((t, _TOP_K), jnp.float32),
        ],
        compiler_params=pltpu_params(),
    )(flat, weight)
    return topi, topw


def pltpu_params():
    from jax.experimental.pallas import tpu as pltpu

    return pltpu.CompilerParams(dimension_semantics=("parallel",))


def kernel(hidden_states, weight):
    bsz, seq_len, h = hidden_states.shape
    flat = hidden_states.reshape(-1, h)
    topi, topw = _gate(flat, weight)
    aux_loss = jnp.float32(0.0)
    return (topi, topw, aux_loss)


# speedup vs baseline: 1.0157x; 1.0157x over previous
import functools

import jax
import jax.numpy as jnp
from jax.experimental import pallas as pl

_TOP_K = 8
_NEG_INF = float("-inf")


_N_SUB = 4


def _gate_body(x_ref, w_ref, idx_ref, wgt_ref):
    w = w_ref[:]
    sb = x_ref.shape[0] // _N_SUB
    for s in range(_N_SUB):
        rows = pl.ds(s * sb, sb)
        logits = jax.lax.dot_general(
            x_ref[rows, :], w, (((1,), (1,)), ((), ())),
            preferred_element_type=jnp.float32,
        )
        topi, wgt = _topk_softmax(logits)
        idx_ref[rows, :] = topi
        wgt_ref[rows, :] = wgt


def _topk_softmax(logits):
    bt, e = logits.shape
    lanef = jax.lax.broadcasted_iota(jnp.int32, (bt, e), 1).astype(jnp.float32)

    vals = []
    idxs = []
    cur = logits
    for _ in range(_TOP_K):
        m = jnp.max(cur, axis=-1, keepdims=True)
        eq = cur == m
        a = jnp.min(
            jnp.where(eq, lanef, jnp.float32(e)), axis=-1, keepdims=True
        )
        vals.append(m)
        idxs.append(a)
        cur = jnp.where(eq, _NEG_INF, cur)

    topv = jnp.concatenate(vals, axis=-1)
    topi = jnp.concatenate(idxs, axis=-1).astype(jnp.int32)

    ex = jnp.exp(topv - topv[:, 0:1])
    wgt = ex / jnp.sum(ex, axis=-1, keepdims=True)
    return topi, wgt


@functools.partial(jax.jit, static_argnames=())
def _gate(flat, weight):
    t, h = flat.shape
    e = weight.shape[0]
    bt = 1024
    grid = (t // bt,)
    topi, topw = pl.pallas_call(
        _gate_body,
        grid=grid,
        in_specs=[
            pl.BlockSpec((bt, h), lambda i: (i, 0)),
            pl.BlockSpec((e, h), lambda i: (0, 0)),
        ],
        out_specs=[
            pl.BlockSpec((bt, _TOP_K), lambda i: (i, 0)),
            pl.BlockSpec((bt, _TOP_K), lambda i: (i, 0)),
        ],
        out_shape=[
            jax.ShapeDtypeStruct((t, _TOP_K), jnp.int32),
            jax.ShapeDtypeStruct((t, _TOP_K), jnp.float32),
        ],
        compiler_params=pltpu_params(),
    )(flat, weight)
    return topi, topw


def pltpu_params():
    from jax.experimental.pallas import tpu as pltpu

    return pltpu.CompilerParams(dimension_semantics=("parallel",))


def kernel(hidden_states, weight):
    bsz, seq_len, h = hidden_states.shape
    flat = hidden_states.reshape(-1, h)
    topi, topw = _gate(flat, weight)
    aux_loss = jnp.float32(0.0)
    return (topi, topw, aux_loss)


# speedup vs baseline: 1.0731x; 1.0565x over previous
import functools

import jax
import jax.numpy as jnp
from jax.experimental import pallas as pl

_TOP_K = 8
_NEG_INF = float("-inf")


_N_SUB = 4


def _gate_body(x_ref, w_ref, idx_ref, wgt_ref):
    w = w_ref[:]
    sb = x_ref.shape[0] // _N_SUB
    for s in range(_N_SUB):
        rows = pl.ds(s * sb, sb)
        logits = jax.lax.dot_general(
            x_ref[rows, :], w, (((1,), (1,)), ((), ())),
            preferred_element_type=jnp.float32,
        )
        topi, wgt = _topk_softmax(logits)
        idx_ref[rows, :] = topi
        wgt_ref[rows, :] = wgt


def _topk_softmax(logits):
    bt, e = logits.shape
    lane = jax.lax.broadcasted_iota(jnp.int32, (bt, e), 1)

    bits = jax.lax.bitcast_convert_type(logits, jnp.int32)
    sign = jax.lax.shift_right_arithmetic(bits, 31)
    tag = jnp.bitwise_xor(jnp.int32(e - 1) - lane, jnp.bitwise_and(sign, 0x3F))
    kbits = jnp.bitwise_or(jnp.bitwise_and(bits, jnp.int32(~0x3F)), tag)
    key = jax.lax.bitcast_convert_type(kbits, jnp.float32)

    keys = []
    cur = key
    for _ in range(_TOP_K):
        m = jnp.max(cur, axis=-1, keepdims=True)
        keys.append(m)
        cur = jnp.where(cur == m, _NEG_INF, cur)

    topk = jnp.concatenate(keys, axis=-1)
    tbits = jax.lax.bitcast_convert_type(topk, jnp.int32)
    tsign = jax.lax.shift_right_arithmetic(tbits, 31)
    ttag = jnp.bitwise_xor(
        jnp.bitwise_and(tbits, jnp.int32(0x3F)), jnp.bitwise_and(tsign, 0x3F)
    )
    topi = jnp.int32(e - 1) - ttag

    topv = jax.lax.bitcast_convert_type(
        jnp.bitwise_and(tbits, jnp.int32(~0x3F)), jnp.float32
    )

    ex = jnp.exp(topv - topv[:, 0:1])
    wgt = ex / jnp.sum(ex, axis=-1, keepdims=True)
    return topi, wgt


@functools.partial(jax.jit, static_argnames=())
def _gate(flat, weight):
    t, h = flat.shape
    e = weight.shape[0]
    bt = 1024
    grid = (t // bt,)
    topi, topw = pl.pallas_call(
        _gate_body,
        grid=grid,
        in_specs=[
            pl.BlockSpec((bt, h), lambda i: (i, 0)),
            pl.BlockSpec((e, h), lambda i: (0, 0)),
        ],
        out_specs=[
            pl.BlockSpec((bt, _TOP_K), lambda i: (i, 0)),
            pl.BlockSpec((bt, _TOP_K), lambda i: (i, 0)),
        ],
        out_shape=[
            jax.ShapeDtypeStruct((t, _TOP_K), jnp.int32),
            jax.ShapeDtypeStruct((t, _TOP_K), jnp.float32),
        ],
        compiler_params=pltpu_params(),
    )(flat, weight)
    return topi, topw


def pltpu_params():
    from jax.experimental.pallas import tpu as pltpu

    return pltpu.CompilerParams(dimension_semantics=("parallel",))


def kernel(hidden_states, weight):
    bsz, seq_len, h = hidden_states.shape
    flat = hidden_states.reshape(-1, h)
    topi, topw = _gate(flat, weight)
    aux_loss = jnp.float32(0.0)
    return (topi, topw, aux_loss)
